# baseline (speedup 1.0000x reference)
import jax
import jax.numpy as jnp
from jax import lax
from jax.experimental import pallas as pl
from jax.experimental.pallas import tpu as pltpu

D = 1000
VP = 1024
R = 512


def _tc_body(idx_ref, tab_ref, out_ref):
    ids = idx_ref[0]
    viota = lax.broadcasted_iota(jnp.int32, (VP, R), 0)
    oht = (viota == ids).astype(jnp.bfloat16)
    out_ref[...] = lax.dot_general(
        oht, tab_ref[...], (((0,), (0,)), ((), ())),
        preferred_element_type=jnp.float32)


def kernel(token_idx, targets, embedding_table):
    B, L = token_idx.shape
    N = B * L
    NB = N // R
    idx = token_idx.reshape(NB, 1, R).astype(jnp.int32)
    tab = jnp.pad(embedding_table.astype(jnp.bfloat16),
                  ((0, VP - embedding_table.shape[0]), (0, 0)))
    out = pl.pallas_call(
        _tc_body,
        grid=(NB,),
        in_specs=[
            pl.BlockSpec((1, 1, R), lambda i: (i, 0, 0)),
            pl.BlockSpec((VP, D), lambda i: (0, 0)),
        ],
        out_specs=pl.BlockSpec((R, D), lambda i: (i, 0)),
        out_shape=jax.ShapeDtypeStruct((N, D), jnp.float32),
    )(idx, tab)
    return out.reshape(B, L, D)


# speedup vs baseline: 1.1224x; 1.1224x over previous
import jax
import jax.numpy as jnp
from jax import lax
from jax.experimental import pallas as pl
from jax.experimental.pallas import tpu as pltpu
from jax.experimental.pallas import tpu_sc as plsc

D = 1000
NC, NS = 2, 16
NW = NC * NS
CHUNK = 16
NBUF = 4


def _gather_body(table_hbm, idx_hbm, out_hbm,
                 table_sh, idx_v, rows0, rows1, rows2, rows3,
                 isem, tsem, gsem0, gsem1, gsem2, gsem3,
                 osem0, osem1, osem2, osem3):
    n_idx = idx_hbm.shape[0]
    b_per_w = n_idx // NW
    n_chunks = b_per_w // CHUNK
    sid = lax.axis_index("s")
    wid = sid * NC + lax.axis_index("c")
    base = wid * b_per_w

    rows = [rows0, rows1, rows2, rows3]
    gsem = [gsem0, gsem1, gsem2, gsem3]
    osem = [osem0, osem1, osem2, osem3]

    def gather(i, k):
        src = table_sh.at[idx_v.at[pl.ds(i * CHUNK, CHUNK)]]
        return pltpu.make_async_copy(src, rows[k], gsem[k])

    def writeback(i, k):
        dst = out_hbm.at[pl.ds(base + i * CHUNK, CHUNK)]
        return pltpu.make_async_copy(rows[k], dst, osem[k])

    pltpu.make_async_copy(idx_hbm.at[pl.ds(base, b_per_w)], idx_v, isem).start()

    @pl.when(sid == 0)
    def _():
        pltpu.make_async_copy(table_hbm, table_sh, tsem).start()
        pltpu.make_async_copy(table_hbm, table_sh, tsem).wait()

    plsc.subcore_barrier()
    pltpu.make_async_copy(idx_hbm.at[pl.ds(base, b_per_w)], idx_v, isem).wait()

    gather(0, 0).start()
    gather(1, 1).start()
    gather(0, 0).wait()
    writeback(0, 0).start()
    gather(2, 2).start()
    gather(1, 1).wait()
    writeback(1, 1).start()
    gather(3, 3).start()

    def group_body(j, carry):
        g0 = 4 * j + 2
        for k in range(4):
            g = g0 + k
            kb = (2 + k) % NBUF
            ko = k
            gather(g, kb).wait()
            writeback(g, kb).start()
            writeback(g - 2, ko).wait()
            gather(g + 2, ko).start()
        return carry

    lax.fori_loop(0, (n_chunks - 4) // 4, group_body, 0)

    for g in (n_chunks - 2, n_chunks - 1):
        gather(g, g % NBUF).wait()
        writeback(g, g % NBUF).start()
        writeback(g - 2, (g - 2) % NBUF).wait()
    writeback(n_chunks - 2, (n_chunks - 2) % NBUF).wait()
    writeback(n_chunks - 1, (n_chunks - 1) % NBUF).wait()


def kernel(token_idx, targets, embedding_table):
    B, L = token_idx.shape
    idx = token_idx.reshape(-1).astype(jnp.int32)
    b_per_w = (B * L) // NW
    mesh = plsc.VectorSubcoreMesh(core_axis_name="c", subcore_axis_name="s")
    out = pl.kernel(
        _gather_body,
        out_type=jax.ShapeDtypeStruct((B * L, D), jnp.float32),
        mesh=mesh,
        compiler_params=pltpu.CompilerParams(use_tc_tiling_on_sc=False),
        scratch_types=[
            pltpu.VMEM_SHARED(embedding_table.shape, jnp.float32),
            pltpu.VMEM((b_per_w,), jnp.int32),
            pltpu.VMEM((CHUNK, D), jnp.float32),
            pltpu.VMEM((CHUNK, D), jnp.float32),
            pltpu.VMEM((CHUNK, D), jnp.float32),
            pltpu.VMEM((CHUNK, D), jnp.float32),
        ] + [pltpu.SemaphoreType.DMA] * 10,
    )(embedding_table, idx)
    return out.reshape(B, L, D)
